# baseline (speedup 1.0000x reference)
import functools

import jax
import jax.numpy as jnp
import numpy as np
from jax.experimental import pallas as pl
from jax.experimental.pallas import tpu as pltpu
from jax.sharding import PartitionSpec as P


def _moe_block_kernel(x_ref, gw_ref, gb_ref, w_ref, b_ref, o_ref, *, n_exp):
    j = pl.program_id(1)
    xb = x_ref[...].astype(jnp.bfloat16)
    w = w_ref[0].astype(jnp.bfloat16)
    y = jax.lax.dot_general(
        xb, w, (((1,), (1,)), ((), ())), preferred_element_type=jnp.float32
    )
    y = jnp.maximum(y + b_ref[0], 0.0)

    @pl.when(j == 0)
    def _():
        o_ref[...] = y

    @pl.when(j > 0)
    def _():
        o_ref[...] += y

    @pl.when(j == n_exp - 1)
    def _():
        gw = gw_ref[...].astype(jnp.bfloat16)
        logits = jax.lax.dot_general(
            xb, gw, (((1,), (1,)), ((), ())), preferred_element_type=jnp.float32
        ) + gb_ref[...]
        p = jax.nn.softmax(logits, axis=-1)
        m1 = jnp.max(p, axis=-1, keepdims=True)
        lane = jax.lax.broadcasted_iota(jnp.int32, p.shape, 1)
        first = jnp.min(
            jnp.where(p == m1, lane, p.shape[-1]), axis=-1, keepdims=True
        )
        m2 = jnp.max(jnp.where(lane == first, -1.0, p), axis=-1, keepdims=True)
        o_ref[...] *= m1 + m2


def _moe_pallas(xf, gate_W, gb2, expert_W, eb3, *, blk):
    T, D = xf.shape
    E = gate_W.shape[0]
    n_tblk = T // blk
    return pl.pallas_call(
        functools.partial(_moe_block_kernel, n_exp=E),
        grid=(n_tblk, E),
        in_specs=[
            pl.BlockSpec((blk, D), lambda t, j: (t, 0)),
            pl.BlockSpec((E, D), lambda t, j: (0, 0)),
            pl.BlockSpec((1, E), lambda t, j: (0, 0)),
            pl.BlockSpec((1, D, D), lambda t, j: (j, 0, 0)),
            pl.BlockSpec((1, 1, D), lambda t, j: (j, 0, 0)),
        ],
        out_specs=pl.BlockSpec((blk, D), lambda t, j: (t, 0)),
        out_shape=jax.ShapeDtypeStruct((T, D), jnp.float32),
        compiler_params=pltpu.CompilerParams(
            dimension_semantics=("parallel", "arbitrary")
        ),
    )(xf, gate_W, gb2, expert_W, eb3)


def kernel(x, gate_W, gate_b, expert_W, expert_b):
    B, S, D = x.shape
    E = gate_W.shape[0]
    T = B * S

    xf = x.reshape(T, D)
    gb2 = gate_b.reshape(1, E)
    eb3 = expert_b.reshape(E, 1, D)

    n_dev = 2 if len(jax.devices()) >= 2 else 1
    blk = 2048
    if n_dev > 1 and T % (blk * n_dev) == 0:
        mesh = jax.sharding.Mesh(np.array(jax.devices()[:n_dev]), ("d",))
        fn = jax.shard_map(
            functools.partial(_moe_pallas, blk=blk),
            mesh=mesh,
            in_specs=(P("d", None), P(None, None), P(None, None),
                      P(None, None, None), P(None, None, None)),
            out_specs=P("d", None),
            check_vma=False,
        )
        out = fn(xf, gate_W, gb2, expert_W, eb3)
    else:
        out = _moe_pallas(xf, gate_W, gb2, expert_W, eb3, blk=blk)
    return out.reshape(B, S, D)


# speedup vs baseline: 4.8514x; 4.8514x over previous
import functools

import jax
import jax.numpy as jnp
from jax.experimental import pallas as pl
from jax.experimental.pallas import tpu as pltpu


def _moe_block_kernel(x_ref, gw_ref, gb_ref, w_ref, b_ref, o_ref, *, n_exp):
    xb = x_ref[...].astype(jnp.bfloat16)

    logits = jax.lax.dot_general(
        xb, gw_ref[...], (((1,), (1,)), ((), ())),
        preferred_element_type=jnp.float32,
    ) + gb_ref[...]
    p = jax.nn.softmax(logits, axis=-1)
    m1 = jnp.max(p, axis=-1, keepdims=True)
    lane = jax.lax.broadcasted_iota(jnp.int32, p.shape, 1)
    first = jnp.min(jnp.where(p == m1, lane, p.shape[-1]), axis=-1, keepdims=True)
    m2 = jnp.max(jnp.where(lane == first, -1.0, p), axis=-1, keepdims=True)
    wsum = m1 + m2

    acc = None
    for j in range(n_exp):
        y = jax.lax.dot_general(
            xb, w_ref[j], (((1,), (1,)), ((), ())),
            preferred_element_type=jnp.float32,
        )
        y = jnp.maximum(y + b_ref[j], 0.0)
        acc = y if acc is None else acc + y
    o_ref[...] = acc * wsum


def _moe_pallas(xf, gw, gb2, ew, eb3, *, blk):
    T, D = xf.shape
    E = gw.shape[0]
    n_tblk = T // blk
    return pl.pallas_call(
        functools.partial(_moe_block_kernel, n_exp=E),
        grid=(n_tblk,),
        in_specs=[
            pl.BlockSpec((blk, D), lambda t: (t, 0)),
            pl.BlockSpec((E, D), lambda t: (0, 0)),
            pl.BlockSpec((1, E), lambda t: (0, 0)),
            pl.BlockSpec((E, D, D), lambda t: (0, 0, 0)),
            pl.BlockSpec((E, 1, D), lambda t: (0, 0, 0)),
        ],
        out_specs=pl.BlockSpec((blk, D), lambda t: (t, 0)),
        out_shape=jax.ShapeDtypeStruct((T, D), jnp.float32),
        compiler_params=pltpu.CompilerParams(
            dimension_semantics=("arbitrary",)
        ),
    )(xf, gw, gb2, ew, eb3)


def kernel(x, gate_W, gate_b, expert_W, expert_b):
    B, S, D = x.shape
    E = gate_W.shape[0]
    T = B * S

    xf = x.reshape(T, D)
    gw = gate_W.astype(jnp.bfloat16)
    gb2 = gate_b.reshape(1, E)
    ew = expert_W.astype(jnp.bfloat16)
    eb3 = expert_b.reshape(E, 1, D)

    out = _moe_pallas(xf, gw, gb2, ew, eb3, blk=512)
    return out.reshape(B, S, D)


# speedup vs baseline: 5.4147x; 1.1161x over previous
import functools

import jax
import jax.numpy as jnp
from jax.experimental import pallas as pl
from jax.experimental.pallas import tpu as pltpu


def _moe_block_kernel(x_ref, gw_ref, gb_ref, w_ref, b_ref, o_ref, *, n_exp):
    xb = x_ref[...].astype(jnp.bfloat16)

    logits = jax.lax.dot_general(
        gw_ref[...], xb, (((1,), (1,)), ((), ())),
        preferred_element_type=jnp.float32,
    ) + gb_ref[...]
    p = jax.nn.softmax(logits, axis=0)
    m1 = jnp.max(p, axis=0, keepdims=True)
    row = jax.lax.broadcasted_iota(jnp.int32, p.shape, 0)
    first = jnp.min(jnp.where(p == m1, row, p.shape[0]), axis=0, keepdims=True)
    m2 = jnp.max(jnp.where(row == first, -1.0, p), axis=0, keepdims=True)
    wsum = (m1 + m2).T

    acc = None
    for j in range(n_exp):
        y = jax.lax.dot_general(
            xb, w_ref[j].astype(jnp.bfloat16), (((1,), (1,)), ((), ())),
            preferred_element_type=jnp.float32,
        )
        y = jnp.maximum(y + b_ref[j], 0.0)
        acc = y if acc is None else acc + y
    o_ref[...] = acc * wsum


def _moe_pallas(xf, gw, gb2, ew, eb3, *, blk):
    T, D = xf.shape
    E = gw.shape[0]
    n_tblk = T // blk
    return pl.pallas_call(
        functools.partial(_moe_block_kernel, n_exp=E),
        grid=(n_tblk,),
        in_specs=[
            pl.BlockSpec((blk, D), lambda t: (t, 0)),
            pl.BlockSpec((E, D), lambda t: (0, 0)),
            pl.BlockSpec((E, 1), lambda t: (0, 0)),
            pl.BlockSpec((E, D, D), lambda t: (0, 0, 0)),
            pl.BlockSpec((E, 1, D), lambda t: (0, 0, 0)),
        ],
        out_specs=pl.BlockSpec((blk, D), lambda t: (t, 0)),
        out_shape=jax.ShapeDtypeStruct((T, D), jnp.float32),
        compiler_params=pltpu.CompilerParams(
            dimension_semantics=("arbitrary",)
        ),
    )(xf, gw, gb2, ew, eb3)


def kernel(x, gate_W, gate_b, expert_W, expert_b):
    B, S, D = x.shape
    E = gate_W.shape[0]
    T = B * S

    xf = x.reshape(T, D)
    gw = gate_W.astype(jnp.bfloat16)
    gb2 = gate_b.reshape(E, 1)
    eb3 = expert_b.reshape(E, 1, D)

    out = _moe_pallas(xf, gw, gb2, expert_W, eb3, blk=512)
    return out.reshape(B, S, D)
